# Initial kernel scaffold; baseline (speedup 1.0000x reference)
import jax
import jax.numpy as jnp
from jax.experimental import pallas as pl


def kernel(relative_position, W):
    raise NotImplementedError("write your pallas kernel here")


# speedup vs baseline: 8.3180x; 8.3180x over previous
import functools
import math

import jax
import jax.numpy as jnp
from jax import lax
from jax.experimental import pallas as pl
from jax.experimental.pallas import tpu as pltpu
from jax.experimental.pallas import tpu_sc as plsc

_NUM_BUCKETS = 64
_MAX_DISTANCE = 256
_OUT_DIM = 16
_SEQ = 2048
_N = _SEQ * _SEQ

_NC = 2
_NS = 16
_NW = _NC * _NS
_PER_W = _N // _NW
_CH = 2048
_NG = _PER_W // _CH
_IDX_ROWS = _CH // 128


def _table_body(w_ref, t_ref):
    half = _NUM_BUCKETS // 2
    max_exact = half // 2
    v = lax.broadcasted_iota(jnp.int32, (_SEQ, _NUM_BUCKETS), 0)
    col = lax.broadcasted_iota(jnp.int32, (_SEQ, _NUM_BUCKETS), 1)
    val_large = max_exact + (
        jnp.log(v / max_exact)
        / math.log(_MAX_DISTANCE / max_exact)
        * (half - max_exact)
    ).astype(jnp.int32)
    val_large = jnp.minimum(val_large, jnp.full_like(val_large, half - 1))
    bucket = jnp.where(v < max_exact, v, val_large)
    onehot = (bucket == col).astype(jnp.float32)
    t_ref[...] = jnp.dot(onehot, w_ref[...], preferred_element_type=jnp.float32)


def _build_table(w):
    return pl.pallas_call(
        _table_body,
        out_shape=jax.ShapeDtypeStruct((_SEQ, _OUT_DIM), jnp.float32),
    )(w)


@functools.partial(
    pl.kernel,
    out_type=jax.ShapeDtypeStruct((_N, _OUT_DIM), jnp.float32),
    mesh=plsc.VectorSubcoreMesh(core_axis_name="c", subcore_axis_name="s"),
    compiler_params=pltpu.CompilerParams(use_tc_tiling_on_sc=False),
    scratch_types=[
        pltpu.VMEM((2, _IDX_ROWS, 128), jnp.int32),
        pltpu.VMEM((2, _CH, _OUT_DIM), jnp.float32),
        pltpu.SemaphoreType.DMA,
        pltpu.SemaphoreType.DMA,
        pltpu.SemaphoreType.DMA,
        pltpu.SemaphoreType.DMA,
    ],
)
def _sc_gather(t_hbm, rp_hbm, out_hbm, idx_v, rows_v, sem_i, sem_g, sem_o0, sem_o1):
    wid = lax.axis_index("s") * _NC + lax.axis_index("c")
    row_base = wid * _PER_W
    idx_base = wid * (_PER_W // 128)
    sem_o = (sem_o0, sem_o1)

    def issue_idx(g, b):
        pltpu.async_copy(
            rp_hbm.at[pl.ds(idx_base + g * _IDX_ROWS, _IDX_ROWS)],
            idx_v.at[b],
            sem_i,
        )

    def wait_idx(b):
        pltpu.make_async_copy(
            rp_hbm.at[pl.ds(0, _IDX_ROWS)], idx_v.at[b], sem_i
        ).wait()

    def issue_gathers(b):
        def one(j, c):
            pltpu.async_copy(
                t_hbm.at[idx_v.at[b, j]],
                rows_v.at[b, pl.ds(j * 128, 128)],
                sem_g,
            )
            return c

        lax.fori_loop(0, _IDX_ROWS, one, 0)

    def wait_gathers(b):
        pltpu.make_async_copy(
            out_hbm.at[pl.ds(0, _CH)], rows_v.at[b], sem_g
        ).wait()

    def issue_store(g, b):
        pltpu.async_copy(
            rows_v.at[b],
            out_hbm.at[pl.ds(row_base + g * _CH, _CH)],
            sem_o[b],
        )

    def wait_store(b):
        pltpu.make_async_copy(
            rows_v.at[b], out_hbm.at[pl.ds(0, _CH)], sem_o[b]
        ).wait()

    issue_idx(0, 0)

    def pair(i, c):
        for b in (0, 1):
            g = i * 2 + b

            @pl.when(g >= 2)
            def _():
                wait_store(b)

            wait_idx(b)

            @pl.when(g + 1 < _NG)
            def _():
                issue_idx(g + 1, 1 - b)

            issue_gathers(b)
            wait_gathers(b)
            issue_store(g, b)
        return c

    lax.fori_loop(0, _NG // 2, pair, 0)
    wait_store(0)
    wait_store(1)


def kernel(relative_position, W):
    t = _build_table(W)
    rp_rows = relative_position.reshape(_N // 128, 128)
    out = _sc_gather(t, rp_rows)
    return out.reshape(_SEQ, _SEQ, _OUT_DIM)


# speedup vs baseline: 13.6743x; 1.6439x over previous
import functools
import math

import jax
import jax.numpy as jnp
from jax import lax
from jax.experimental import pallas as pl
from jax.experimental.pallas import tpu as pltpu
from jax.experimental.pallas import tpu_sc as plsc

_NUM_BUCKETS = 64
_MAX_DISTANCE = 256
_OUT_DIM = 16
_SEQ = 2048

_NC = 2
_NS = 16
_NW = _NC * _NS
_ROWS_W = _SEQ // _NW
_NT = _SEQ // 128


def _table_body(w_ref, t_ref):
    half = _NUM_BUCKETS // 2
    max_exact = half // 2
    v = lax.broadcasted_iota(jnp.int32, (_SEQ, _NUM_BUCKETS), 0)
    col = lax.broadcasted_iota(jnp.int32, (_SEQ, _NUM_BUCKETS), 1)
    val_large = max_exact + (
        jnp.log(v / max_exact)
        / math.log(_MAX_DISTANCE / max_exact)
        * (half - max_exact)
    ).astype(jnp.int32)
    val_large = jnp.minimum(val_large, jnp.full_like(val_large, half - 1))
    bucket = jnp.where(v < max_exact, v, val_large)
    onehot = (bucket == col).astype(jnp.float32)
    t_ref[...] = jnp.dot(onehot, w_ref[...], preferred_element_type=jnp.float32)


def _build_table(w):
    return pl.pallas_call(
        _table_body,
        out_shape=jax.ShapeDtypeStruct((_SEQ, _OUT_DIM), jnp.float32),
    )(w)


@functools.partial(
    pl.kernel,
    out_type=jax.ShapeDtypeStruct((_SEQ, 2 * _NT, 8, 128), jnp.float32),
    mesh=plsc.VectorSubcoreMesh(core_axis_name="c", subcore_axis_name="s"),
    compiler_params=pltpu.CompilerParams(
        use_tc_tiling_on_sc=True, needs_layout_passes=False
    ),
    scratch_types=[
        pltpu.VMEM((32, 8, 128), jnp.float32),
        pltpu.VMEM((1, _NT, 8, 128), jnp.int32),
        pltpu.VMEM((2, 1, 2 * _NT, 8, 128), jnp.float32),
        pltpu.SemaphoreType.DMA,
        pltpu.SemaphoreType.DMA,
        pltpu.SemaphoreType.DMA,
    ],
)
def _sc_gather(t_hbm, rp_hbm, out_hbm, t_v, idx_v, obuf, sem_i, sem_o0, sem_o1):
    wid = lax.axis_index("s") * _NC + lax.axis_index("c")
    tile_row0 = wid * (_ROWS_W // 8)
    sem_o = (sem_o0, sem_o1)

    def wait_store(b):
        pltpu.make_async_copy(
            obuf.at[b], out_hbm.at[pl.ds(0, 1)], sem_o[b]
        ).wait()

    pltpu.sync_copy(t_hbm, t_v)

    def chunk(a, c):
        it = tile_row0 + a
        pltpu.async_copy(rp_hbm.at[pl.ds(it, 1)], idx_v, sem_i)
        pltpu.make_async_copy(rp_hbm.at[pl.ds(0, 1)], idx_v, sem_i).wait()

        def pair(p, cc):
            for b in (0, 1):
                r = p * 2 + b

                @pl.when(a * 8 + r >= 2)
                def _():
                    wait_store(b)

                def jtile(jt, c2):
                    for c8 in range(8):
                        jvec = idx_v[0, jt, r, pl.ds(c8 * 16, 16)]
                        rt = jvec >> 6
                        rr = (jvec >> 3) & 7
                        colb = (jvec & 7) << 4
                        for d in range(_OUT_DIM):
                            g = plsc.load_gather(t_v, [rt, rr, colb + d])
                            obuf[b, 0, (d // 8) * _NT + jt, d % 8, pl.ds(c8 * 16, 16)] = g
                    return c2

                lax.fori_loop(0, _NT, jtile, 0)
                pltpu.async_copy(
                    obuf.at[b], out_hbm.at[pl.ds(it * 8 + r, 1)], sem_o[b]
                )
            return cc

        lax.fori_loop(0, 4, pair, 0)
        return c

    lax.fori_loop(0, _ROWS_W // 8, chunk, 0)
    wait_store(0)
    wait_store(1)


def kernel(relative_position, W):
    t = _build_table(W)
    t3 = t.reshape(32, 8, 128)
    rp4 = relative_position.reshape(_SEQ // 8, 8, _NT, 128).transpose(0, 2, 1, 3)
    out4 = _sc_gather(t3, rp4)
    out = (
        out4.reshape(_SEQ, 2, _NT, 8, 128)
        .transpose(0, 2, 4, 1, 3)
        .reshape(_SEQ, _SEQ, _OUT_DIM)
    )
    return out


# speedup vs baseline: 33.8057x; 2.4722x over previous
import functools
import math

import jax
import jax.numpy as jnp
from jax import lax
from jax.experimental import pallas as pl
from jax.experimental.pallas import tpu as pltpu
from jax.experimental.pallas import tpu_sc as plsc

_NUM_BUCKETS = 64
_MAX_DISTANCE = 256
_OUT_DIM = 16
_SEQ = 2048

_NC = 2
_NS = 16
_NW = _NC * _NS
_ROWS_W = _SEQ // _NW
_NT = _SEQ // 128


def _table_body(w_ref, t_ref):
    half = _NUM_BUCKETS // 2
    max_exact = half // 2
    v = lax.broadcasted_iota(jnp.int32, (_SEQ, _NUM_BUCKETS), 0)
    col = lax.broadcasted_iota(jnp.int32, (_SEQ, _NUM_BUCKETS), 1)
    val_large = max_exact + (
        jnp.log(v / max_exact)
        / math.log(_MAX_DISTANCE / max_exact)
        * (half - max_exact)
    ).astype(jnp.int32)
    val_large = jnp.minimum(val_large, jnp.full_like(val_large, half - 1))
    bucket = jnp.where(v < max_exact, v, val_large)
    onehot = (bucket == col).astype(jnp.float32)
    t_ref[...] = jnp.dot(onehot, w_ref[...], preferred_element_type=jnp.float32)


def _build_table(w):
    return pl.pallas_call(
        _table_body,
        out_shape=jax.ShapeDtypeStruct((_SEQ, _OUT_DIM), jnp.float32),
    )(w)


@functools.partial(
    pl.kernel,
    out_type=jax.ShapeDtypeStruct((_SEQ, 2 * _NT, 8, 128), jnp.float32),
    mesh=plsc.VectorSubcoreMesh(core_axis_name="c", subcore_axis_name="s"),
    compiler_params=pltpu.CompilerParams(
        use_tc_tiling_on_sc=True, needs_layout_passes=False
    ),
    scratch_types=[
        pltpu.VMEM((32, 8, 128), jnp.float32),
        pltpu.VMEM((1, _NT, 8, 128), jnp.int32),
        pltpu.VMEM((2, 1, 2 * _NT, 8, 128), jnp.float32),
        pltpu.SemaphoreType.DMA,
        pltpu.SemaphoreType.DMA,
        pltpu.SemaphoreType.DMA,
    ],
)
def _sc_gather(t_hbm, rp_hbm, out_hbm, t_v, idx_v, obuf, sem_i, sem_o0, sem_o1):
    wid = lax.axis_index("s") * _NC + lax.axis_index("c")
    tile_row0 = wid * (_ROWS_W // 8)
    sem_o = (sem_o0, sem_o1)

    def wait_store(b):
        pltpu.make_async_copy(
            obuf.at[b], out_hbm.at[pl.ds(0, 1)], sem_o[b]
        ).wait()

    pltpu.sync_copy(t_hbm, t_v)

    def chunk(a, c):
        it = tile_row0 + a
        pltpu.async_copy(rp_hbm.at[pl.ds(it, 1)], idx_v, sem_i)
        pltpu.make_async_copy(rp_hbm.at[pl.ds(0, 1)], idx_v, sem_i).wait()

        def pair(p, cc):
            for b in (0, 1):
                r = p * 2 + b

                @pl.when(a * 8 + r >= 2)
                def _():
                    wait_store(b)

                @plsc.parallel_loop(0, _NT, 1, unroll=2)
                def _(jt):
                    for c8 in range(8):
                        jvec = idx_v[0, jt, r, pl.ds(c8 * 16, 16)]
                        rt = jvec >> 6
                        rr = (jvec >> 3) & 7
                        colb = (jvec & 7) << 4
                        for d in range(_OUT_DIM):
                            g = plsc.load_gather(t_v, [rt, rr, colb + d])
                            obuf[b, 0, (d // 8) * _NT + jt, d % 8, pl.ds(c8 * 16, 16)] = g
                pltpu.async_copy(
                    obuf.at[b], out_hbm.at[pl.ds(it * 8 + r, 1)], sem_o[b]
                )
            return cc

        lax.fori_loop(0, 4, pair, 0)
        return c

    lax.fori_loop(0, _ROWS_W // 8, chunk, 0)
    wait_store(0)
    wait_store(1)


def kernel(relative_position, W):
    t = _build_table(W)
    t3 = t.reshape(32, 8, 128)
    rp4 = relative_position.reshape(_SEQ // 8, 8, _NT, 128).transpose(0, 2, 1, 3)
    out4 = _sc_gather(t3, rp4)
    out = (
        out4.reshape(_SEQ, 2, _NT, 8, 128)
        .transpose(0, 2, 4, 1, 3)
        .reshape(_SEQ, _SEQ, _OUT_DIM)
    )
    return out


# speedup vs baseline: 39.7108x; 1.1747x over previous
import functools
import math

import jax
import jax.numpy as jnp
from jax import lax
from jax.experimental import pallas as pl
from jax.experimental.pallas import tpu as pltpu
from jax.experimental.pallas import tpu_sc as plsc

_NUM_BUCKETS = 64
_MAX_DISTANCE = 256
_OUT_DIM = 16
_SEQ = 2048

_NC = 2
_NS = 16
_NW = _NC * _NS
_ROWS_W = _SEQ // _NW
_NT = _SEQ // 128


def _table_body(w_ref, t_ref):
    half = _NUM_BUCKETS // 2
    max_exact = half // 2
    v = lax.broadcasted_iota(jnp.int32, (_SEQ, _NUM_BUCKETS), 0)
    col = lax.broadcasted_iota(jnp.int32, (_SEQ, _NUM_BUCKETS), 1)
    val_large = max_exact + (
        jnp.log(v / max_exact)
        / math.log(_MAX_DISTANCE / max_exact)
        * (half - max_exact)
    ).astype(jnp.int32)
    val_large = jnp.minimum(val_large, jnp.full_like(val_large, half - 1))
    bucket = jnp.where(v < max_exact, v, val_large)
    onehot = (bucket == col).astype(jnp.float32)
    t_ref[...] = jnp.dot(onehot, w_ref[...], preferred_element_type=jnp.float32)


def _build_table(w):
    return pl.pallas_call(
        _table_body,
        out_shape=jax.ShapeDtypeStruct((_SEQ, _OUT_DIM), jnp.float32),
    )(w)


@functools.partial(
    pl.kernel,
    out_type=jax.ShapeDtypeStruct((_SEQ, 2 * _NT, 8, 128), jnp.float32),
    mesh=plsc.VectorSubcoreMesh(core_axis_name="c", subcore_axis_name="s"),
    compiler_params=pltpu.CompilerParams(
        use_tc_tiling_on_sc=True, needs_layout_passes=False
    ),
    scratch_types=[
        pltpu.VMEM((32, 8, 128), jnp.float32),
        pltpu.VMEM((1, _NT, 8, 128), jnp.int32),
        pltpu.VMEM((2, 1, 2 * _NT, 8, 128), jnp.float32),
        pltpu.SemaphoreType.DMA,
        pltpu.SemaphoreType.DMA,
        pltpu.SemaphoreType.DMA,
    ],
)
def _sc_gather(t_hbm, rp_hbm, out_hbm, t_v, idx_v, obuf, sem_i, sem_o0, sem_o1):
    wid = lax.axis_index("s") * _NC + lax.axis_index("c")
    tile_row0 = wid * (_ROWS_W // 8)
    sem_o = (sem_o0, sem_o1)

    def wait_store(b):
        pltpu.make_async_copy(
            obuf.at[b], out_hbm.at[pl.ds(0, 1)], sem_o[b]
        ).wait()

    pltpu.sync_copy(t_hbm, t_v)

    def chunk(a, c):
        it = tile_row0 + a
        pltpu.async_copy(rp_hbm.at[pl.ds(it, 1)], idx_v, sem_i)
        pltpu.make_async_copy(rp_hbm.at[pl.ds(0, 1)], idx_v, sem_i).wait()

        def pair(p, cc):
            for b in (0, 1):
                r = p * 2 + b

                @pl.when(a * 8 + r >= 2)
                def _():
                    wait_store(b)

                @plsc.parallel_loop(0, _SEQ // 16, 1, unroll=2)
                def _(jv):
                    jt = jv >> 3
                    c8 = (jv & 7) << 4
                    jvec = idx_v[0, jt, r, pl.ds(c8, 16)]
                    rt = jvec >> 6
                    rr = (jvec >> 3) & 7
                    colb = (jvec & 7) << 4
                    for d in range(_OUT_DIM):
                        g = plsc.load_gather(t_v, [rt, rr, colb + d])
                        obuf[b, 0, (d // 8) * _NT + jt, d % 8, pl.ds(c8, 16)] = g
                pltpu.async_copy(
                    obuf.at[b], out_hbm.at[pl.ds(it * 8 + r, 1)], sem_o[b]
                )
            return cc

        lax.fori_loop(0, 4, pair, 0)
        return c

    lax.fori_loop(0, _ROWS_W // 8, chunk, 0)
    wait_store(0)
    wait_store(1)


def kernel(relative_position, W):
    t = _build_table(W)
    t3 = t.reshape(32, 8, 128)
    rp4 = relative_position.reshape(_SEQ // 8, 8, _NT, 128).transpose(0, 2, 1, 3)
    out4 = _sc_gather(t3, rp4)
    out = (
        out4.reshape(_SEQ, 2, _NT, 8, 128)
        .transpose(0, 2, 4, 1, 3)
        .reshape(_SEQ, _SEQ, _OUT_DIM)
    )
    return out
